# baseline (speedup 1.0000x reference)
import functools

import jax
import jax.numpy as jnp
from jax import lax
from jax.experimental import pallas as pl
from jax.experimental.pallas import tpu as pltpu
from jax.experimental.pallas import tpu_sc as plsc

EMBED_DIM = 64
NUM_CTX = 5
BATCH = 16384
LANES = 16
NUM_CORES = 2
NUM_SUBCORES = 16
NUM_WORKERS = NUM_CORES * NUM_SUBCORES
SAMPLES_PER_WORKER = BATCH // NUM_WORKERS
CHUNK = 128
NUM_CHUNKS = SAMPLES_PER_WORKER // CHUNK
DREGS = EMBED_DIM // LANES

_mesh = plsc.VectorSubcoreMesh(core_axis_name="c", subcore_axis_name="s")


@functools.partial(
    pl.kernel,
    mesh=_mesh,
    compiler_params=pltpu.CompilerParams(
        needs_layout_passes=False, use_tc_tiling_on_sc=False
    ),
    out_type=jax.ShapeDtypeStruct((BATCH * NUM_CTX,), jnp.float32),
    scratch_types=[
        pltpu.VMEM((CHUNK,), jnp.int32),
        pltpu.VMEM((NUM_CTX, CHUNK), jnp.int32),
        pltpu.VMEM((CHUNK, EMBED_DIM), jnp.float32),
        pltpu.VMEM((NUM_CTX * CHUNK, EMBED_DIM), jnp.float32),
        pltpu.VMEM((NUM_CTX * CHUNK,), jnp.float32),
        pltpu.SemaphoreType.DMA,
    ],
)
def _w2v_sc(target_hbm, context_hbm, ttab_hbm, ctab_hbm, out_hbm,
            tidx_v, cidx_v, we_v, ce_v, out_v, sem):
    wid = lax.axis_index("s") * NUM_CORES + lax.axis_index("c")
    lane = lax.iota(jnp.int32, LANES)
    last = lane == (LANES - 1)

    for j in range(NUM_CHUNKS):
        row0 = pl.multiple_of(wid * SAMPLES_PER_WORKER + j * CHUNK, CHUNK)
        pair0 = pl.multiple_of(row0 * NUM_CTX, CHUNK * NUM_CTX)

        pltpu.sync_copy(target_hbm.at[pl.ds(row0, CHUNK)], tidx_v)
        for i in range(NUM_CTX):
            pltpu.sync_copy(
                context_hbm.at[pl.ds(pair0 + i * CHUNK, CHUNK)], cidx_v.at[i]
            )

        copies = [pltpu.async_copy(ttab_hbm.at[tidx_v], we_v, sem)]
        for i in range(NUM_CTX):
            copies.append(
                pltpu.async_copy(
                    ctab_hbm.at[cidx_v.at[i]],
                    ce_v.at[pl.ds(i * CHUNK, CHUNK), :],
                    sem,
                )
            )
        for c in copies:
            c.wait()

        def body(s, carry):
            wes = [we_v[s, pl.ds(k * LANES, LANES)] for k in range(DREGS)]
            for n in range(NUM_CTX):
                p = s * NUM_CTX + n
                acc = ce_v[p, pl.ds(0, LANES)] * wes[0]
                for k in range(1, DREGS):
                    acc = acc + ce_v[p, pl.ds(k * LANES, LANES)] * wes[k]
                total = plsc.cumsum(acc)
                plsc.store_scatter(
                    out_v, [jnp.full((LANES,), 0, jnp.int32) + p], total, mask=last
                )
            return carry

        lax.fori_loop(0, CHUNK, body, 0)

        pltpu.sync_copy(out_v, out_hbm.at[pl.ds(pair0, NUM_CTX * CHUNK)])


def kernel(target, context, target_table, context_table):
    t = target.reshape(-1).astype(jnp.int32)
    c = context.reshape(-1).astype(jnp.int32)
    flat = _w2v_sc(t, c, target_table, context_table)
    return flat.reshape(BATCH, NUM_CTX)
